# baseline (speedup 1.0000x reference)
import jax
import jax.numpy as jnp
from jax.experimental import pallas as pl

_EPS = 1e-8


def _fear_kernel(x_ref, w1_ref, b1_ref, w2_ref, b2_ref, fe_ref, o_ref):
    x = x_ref[...]
    xb = x.astype(jnp.bfloat16)
    feb = fe_ref[...].astype(jnp.bfloat16)

    h = jax.lax.dot_general(
        xb, w1_ref[...].astype(jnp.bfloat16), (((1,), (1,)), ((), ())),
        preferred_element_type=jnp.float32)
    h = jnp.maximum(h + b1_ref[...], 0.0)
    logits = jax.lax.dot_general(
        h.astype(jnp.bfloat16), w2_ref[...].astype(jnp.bfloat16),
        (((1,), (1,)), ((), ())), preferred_element_type=jnp.float32)
    logits = logits + b2_ref[...]
    m = jnp.max(logits, axis=-1, keepdims=True)
    ex = jnp.exp(logits - m)
    context = ex / jnp.sum(ex, axis=-1, keepdims=True)
    cb = context.astype(jnp.bfloat16)

    xfe = jax.lax.dot_general(
        xb, feb, (((1,), (1,)), ((), ())),
        preferred_element_type=jnp.float32)
    C = cb.shape[1]
    num_f = jnp.sum(context * xfe[:, :C], axis=-1)
    num_e = jnp.sum(context * xfe[:, C:], axis=-1)

    gram = jax.lax.dot_general(
        feb, feb, (((1,), (1,)), ((), ())),
        preferred_element_type=jnp.float32)
    cgf = jax.lax.dot_general(
        cb, gram[:C, :C].astype(jnp.bfloat16), (((1,), (1,)), ((), ())),
        preferred_element_type=jnp.float32)
    cge = jax.lax.dot_general(
        cb, gram[C:, C:].astype(jnp.bfloat16), (((1,), (1,)), ((), ())),
        preferred_element_type=jnp.float32)
    nsq_f = jnp.sum(context * cgf, axis=-1)
    nsq_e = jnp.sum(context * cge, axis=-1)

    x_norm = jnp.maximum(jnp.sqrt(jnp.sum(x * x, axis=-1)), _EPS)
    nf = jnp.maximum(jnp.sqrt(jnp.maximum(nsq_f, 0.0)), _EPS)
    ne = jnp.maximum(jnp.sqrt(jnp.maximum(nsq_e, 0.0)), _EPS)
    sim = num_f / (x_norm * nf) - num_e / (x_norm * ne)
    o_ref[...] = jax.nn.sigmoid(sim)[:, None]


@jax.jit
def kernel(sensory_features, W1, b1, W2, b2, fear_memory, extinction_memory):
    B, D = sensory_features.shape
    H = W1.shape[0]
    C = W2.shape[0]
    BB = 1024

    fe = jnp.concatenate([fear_memory, extinction_memory], axis=0)

    rep = lambda i: (0, 0)
    out = pl.pallas_call(
        _fear_kernel,
        grid=(B // BB,),
        in_specs=[
            pl.BlockSpec((BB, D), lambda i: (i, 0)),
            pl.BlockSpec((H, D), rep),
            pl.BlockSpec((1, H), rep),
            pl.BlockSpec((C, H), rep),
            pl.BlockSpec((1, C), rep),
            pl.BlockSpec((2 * C, D), rep),
        ],
        out_specs=pl.BlockSpec((BB, 1), lambda i: (i, 0)),
        out_shape=jax.ShapeDtypeStruct((B, 1), jnp.float32),
    )(sensory_features, W1, b1.reshape(1, H), W2, b2.reshape(1, C), fe)
    return out


# speedup vs baseline: 1.3572x; 1.3572x over previous
import jax
import jax.numpy as jnp
from jax.experimental import pallas as pl

_EPS = 1e-8


def _fear_kernel(x_ref, w1_ref, b1_ref, w2_ref, b2_ref, fe_ref, o_ref):
    x = x_ref[...]
    xb = x.astype(jnp.bfloat16)
    feb = fe_ref[...].astype(jnp.bfloat16)
    C = w2_ref.shape[0]
    D = x.shape[1]

    h = jax.lax.dot_general(
        xb, w1_ref[...].astype(jnp.bfloat16), (((1,), (1,)), ((), ())),
        preferred_element_type=jnp.float32)
    h = jnp.maximum(h + b1_ref[...], 0.0)
    logits = jax.lax.dot_general(
        h.astype(jnp.bfloat16), w2_ref[...].astype(jnp.bfloat16),
        (((1,), (1,)), ((), ())), preferred_element_type=jnp.float32)
    logits = logits + b2_ref[...]
    e = jnp.exp(logits - jnp.max(logits, axis=-1, keepdims=True))

    xfe = jax.lax.dot_general(
        xb, feb, (((1,), (1,)), ((), ())),
        preferred_element_type=jnp.float32)

    gram = jax.lax.dot_general(
        feb, feb, (((1,), (1,)), ((), ())),
        preferred_element_type=jnp.float32)
    gcat = jnp.concatenate([gram[:C, :C], gram[C:, C:]], axis=1)
    eb = e.astype(jnp.bfloat16)
    cg = jax.lax.dot_general(
        eb, gcat.astype(jnp.bfloat16), (((1,), (0,)), ((), ())),
        preferred_element_type=jnp.float32)

    prod = jnp.concatenate([xfe, cg], axis=1) * jnp.concatenate(
        [e, e, e, e], axis=1)
    rows = jax.lax.broadcasted_iota(jnp.int32, (4 * C, 8), 0)
    cols = jax.lax.broadcasted_iota(jnp.int32, (4 * C, 8), 1)
    sel = (rows // C == cols).astype(jnp.bfloat16)
    red = jax.lax.dot_general(
        prod.astype(jnp.bfloat16), sel, (((1,), (0,)), ((), ())),
        preferred_element_type=jnp.float32)

    xsq = xb * xb
    xn2 = jax.lax.dot_general(
        xsq, jnp.ones((D, 8), jnp.bfloat16), (((1,), (0,)), ((), ())),
        preferred_element_type=jnp.float32)

    x_norm = jnp.maximum(jnp.sqrt(xn2[:, 0]), _EPS)
    nf = jnp.maximum(jnp.sqrt(jnp.maximum(red[:, 2], 0.0)), _EPS)
    ne = jnp.maximum(jnp.sqrt(jnp.maximum(red[:, 3], 0.0)), _EPS)
    sim = red[:, 0] / (x_norm * nf) - red[:, 1] / (x_norm * ne)
    o_ref[...] = jax.nn.sigmoid(sim)[:, None]


@jax.jit
def kernel(sensory_features, W1, b1, W2, b2, fear_memory, extinction_memory):
    B, D = sensory_features.shape
    H = W1.shape[0]
    C = W2.shape[0]
    BB = 1024

    fe = jnp.concatenate([fear_memory, extinction_memory], axis=0)

    rep = lambda i: (0, 0)
    out = pl.pallas_call(
        _fear_kernel,
        grid=(B // BB,),
        in_specs=[
            pl.BlockSpec((BB, D), lambda i: (i, 0)),
            pl.BlockSpec((H, D), rep),
            pl.BlockSpec((1, H), rep),
            pl.BlockSpec((C, H), rep),
            pl.BlockSpec((1, C), rep),
            pl.BlockSpec((2 * C, D), rep),
        ],
        out_specs=pl.BlockSpec((BB, 1), lambda i: (i, 0)),
        out_shape=jax.ShapeDtypeStruct((B, 1), jnp.float32),
    )(sensory_features, W1, b1.reshape(1, H), W2, b2.reshape(1, C), fe)
    return out


# speedup vs baseline: 1.7304x; 1.2749x over previous
import jax
import jax.numpy as jnp
from jax.experimental import pallas as pl

_EPS = 1e-8


def _fear_kernel(x_ref, wfe_ref, b1_ref, w2_ref, b2_ref, o_ref):
    x = x_ref[...]
    xb = x.astype(jnp.bfloat16)
    wfb = wfe_ref[...].astype(jnp.bfloat16)
    H = b1_ref.shape[1]
    C = w2_ref.shape[0]
    D = x.shape[1]

    big = jax.lax.dot_general(
        xb, wfb, (((1,), (1,)), ((), ())),
        preferred_element_type=jnp.float32)
    h = jnp.maximum(big[:, :H] + b1_ref[...], 0.0)
    xfe = big[:, H:]

    logits = jax.lax.dot_general(
        h.astype(jnp.bfloat16), w2_ref[...].astype(jnp.bfloat16),
        (((1,), (1,)), ((), ())), preferred_element_type=jnp.float32)
    logits = logits + b2_ref[...]
    e = jnp.exp(logits - jnp.max(logits, axis=-1, keepdims=True))
    eb = e.astype(jnp.bfloat16)

    feb = wfb[H:, :]
    gram = jax.lax.dot_general(
        feb, feb, (((1,), (1,)), ((), ())),
        preferred_element_type=jnp.float32)
    gcat = jnp.concatenate([gram[:C, :C], gram[C:, C:]], axis=1)
    cg = jax.lax.dot_general(
        eb, gcat.astype(jnp.bfloat16), (((1,), (0,)), ((), ())),
        preferred_element_type=jnp.float32)

    ee = jnp.concatenate([e, e], axis=1)
    m_ab = (xfe * ee).astype(jnp.bfloat16)
    m_cg = (cg * ee).astype(jnp.bfloat16)
    xsq = xb * xb

    r_ab = jax.lax.broadcasted_iota(jnp.int32, (8, 2 * C), 0)
    c_ab = jax.lax.broadcasted_iota(jnp.int32, (8, 2 * C), 1)
    sel_ab = (r_ab == c_ab // C).astype(jnp.bfloat16)
    sel_cg = (r_ab == 2 + c_ab // C).astype(jnp.bfloat16)
    r_x = jax.lax.broadcasted_iota(jnp.int32, (8, D), 0)
    sel_x = (r_x == 4).astype(jnp.bfloat16)

    tn = (((1,), (1,)), ((), ()))
    redT = (
        jax.lax.dot_general(sel_ab, m_ab, tn,
                            preferred_element_type=jnp.float32)
        + jax.lax.dot_general(sel_cg, m_cg, tn,
                              preferred_element_type=jnp.float32)
        + jax.lax.dot_general(sel_x, xsq, tn,
                              preferred_element_type=jnp.float32)
    )

    x_norm = jnp.maximum(jnp.sqrt(redT[4:5]), _EPS)
    nf = jnp.maximum(jnp.sqrt(jnp.maximum(redT[2:3], 0.0)), _EPS)
    ne = jnp.maximum(jnp.sqrt(jnp.maximum(redT[3:4], 0.0)), _EPS)
    sim = redT[0:1] / (x_norm * nf) - redT[1:2] / (x_norm * ne)
    o_ref[...] = jax.nn.sigmoid(sim)[None]


@jax.jit
def kernel(sensory_features, W1, b1, W2, b2, fear_memory, extinction_memory):
    B, D = sensory_features.shape
    H = W1.shape[0]
    C = W2.shape[0]
    BB = 1024

    wfe = jnp.concatenate([W1, fear_memory, extinction_memory], axis=0)

    rep = lambda i: (0, 0)
    out = pl.pallas_call(
        _fear_kernel,
        grid=(B // BB,),
        in_specs=[
            pl.BlockSpec((BB, D), lambda i: (i, 0)),
            pl.BlockSpec((H + 2 * C, D), rep),
            pl.BlockSpec((1, H), rep),
            pl.BlockSpec((C, H), rep),
            pl.BlockSpec((1, C), rep),
        ],
        out_specs=pl.BlockSpec((1, 1, BB), lambda i: (i, 0, 0)),
        out_shape=jax.ShapeDtypeStruct((B // BB, 1, BB), jnp.float32),
    )(sensory_features, wfe, b1.reshape(1, H), W2, b2.reshape(1, C))
    return out.reshape(B, 1)
